# baseline (speedup 1.0000x reference)
import functools

import jax
import jax.numpy as jnp
from jax import lax
from jax.experimental import pallas as pl
from jax.experimental.pallas import tpu as pltpu
from jax.experimental.pallas import tpu_sc as plsc

_N = 16777216
_NB = 2097152
_NC = 2
_NS = 16
_NW = _NC * _NS
_K = 1024
_PAD = 32
_PER_W = _NB // _NW
_CHUNKS = _PER_W // _K
_ROUNDS = 6
_RPC = _K // 128
_NBUF = 4


def _muladd_body(lam_ref, m_ref, c_ref, o_ref):
    o_ref[...] = m_ref[...] * lam_ref[0] + c_ref[...]


def _values_tc(lam, m, c):
    nblk = 8
    return pl.pallas_call(
        _muladd_body,
        grid=(nblk,),
        in_specs=[
            pl.BlockSpec(memory_space=pltpu.SMEM),
            pl.BlockSpec((_NB // nblk,), lambda i: (i,)),
            pl.BlockSpec((_NB // nblk,), lambda i: (i,)),
        ],
        out_specs=pl.BlockSpec((_NB // nblk,), lambda i: (i,)),
        out_shape=jax.ShapeDtypeStruct((_NB,), jnp.float32),
    )(lam, m, c)


def _copy_body(x_ref, o_ref):
    o_ref[...] = x_ref[...]


def _copy_tc(q):
    nblk = 16
    return pl.pallas_call(
        _copy_body,
        grid=(nblk,),
        in_specs=[pl.BlockSpec((_N // nblk,), lambda i: (i,))],
        out_specs=pl.BlockSpec((_N // nblk,), lambda i: (i,)),
        out_shape=jax.ShapeDtypeStruct((_N,), jnp.float32),
    )(q)


def _sc_body(si_e, sv_e, out, kbuf, vbuf, sem_in, sem_sc):
    cid = lax.axis_index("c")
    sid = lax.axis_index("s")
    wid = sid * _NC + cid
    base = wid * _PER_W

    _B = _K + _PAD

    def stage(t, b):
        pos = pl.multiple_of(base + t * _K, _K)
        off = pl.multiple_of(b * _B, _B)
        return (
            pltpu.async_copy(si_e.at[pl.ds(pos, _B)], kbuf.at[pl.ds(off, _B)], sem_in),
            pltpu.async_copy(sv_e.at[pl.ds(pos, _B)], vbuf.at[pl.ds(off, _B)], sem_in),
        )

    def wait_stage(b):
        off = pl.multiple_of(b * _B, _B)
        pltpu.make_async_copy(si_e.at[pl.ds(0, _B)], kbuf.at[pl.ds(off, _B)], sem_in).wait()
        pltpu.make_async_copy(sv_e.at[pl.ds(0, _B)], vbuf.at[pl.ds(off, _B)], sem_in).wait()

    def wait_scatter(b):
        off = pl.multiple_of(b * _B, _B)
        pltpu.make_async_copy(
            vbuf.at[pl.ds(off, _K)],
            out.at[kbuf.at[pl.ds(off, _K)]], sem_sc).wait()

    def compute_and_fire(b):
        off = pl.multiple_of(b * _B, _B)
        for _ in range(_ROUNDS):
            for g in range((_K + 16) // 16):
                o = g * 16
                k = kbuf[pl.ds(off + o, 16)]
                kn = kbuf[pl.ds(off + o + 1, 16)]
                v = vbuf[pl.ds(off + o, 16)]
                vn = vbuf[pl.ds(off + o + 1, 16)]
                vbuf[pl.ds(off + o, 16)] = jnp.where(k != kn, v, vn)
        pltpu.async_copy(
            vbuf.at[pl.ds(off, _K)],
            out.at[kbuf.at[pl.ds(off, _K)]], sem_sc)

    stage(0, 0)

    def chunk(t, carry):
        b = lax.rem(t, _NBUF)
        nb = lax.rem(t + 1, _NBUF)

        @pl.when(t >= _NBUF - 1)
        def _():
            wait_scatter(nb)

        @pl.when(t < _CHUNKS - 1)
        def _():
            stage(t + 1, nb)

        wait_stage(b)
        compute_and_fire(b)
        return carry

    lax.fori_loop(0, _CHUNKS, chunk, 0)
    for j in range(_NBUF - 1):
        wait_scatter((_CHUNKS - (_NBUF - 1) + j) % _NBUF)


def _sc_scatter(si_e, sv_e, out_ref):
    mesh = plsc.VectorSubcoreMesh(
        core_axis_name="c", subcore_axis_name="s",
        num_cores=_NC, num_subcores=_NS)
    k = pl.kernel(
        _sc_body,
        out_type=(),
        mesh=mesh,
        scratch_types=[
            pltpu.VMEM((_NBUF * (_K + _PAD),), jnp.int32),
            pltpu.VMEM((_NBUF * (_K + _PAD),), jnp.float32),
            pltpu.SemaphoreType.DMA,
            pltpu.SemaphoreType.DMA,
        ],
    )
    k(si_e, sv_e, out_ref)


def kernel(q, _lambda, idx_b, xb_m, xb_c):
    idx = idx_b.astype(jnp.int32)
    vals = _values_tc(_lambda, xb_m, xb_c)
    si, sv = lax.sort_key_val(idx, vals, is_stable=False)
    si_e = jnp.concatenate([si, jnp.full((_PAD,), -1, jnp.int32)])
    sv_e = jnp.concatenate([sv, jnp.zeros((_PAD,), jnp.float32)])
    out0 = _copy_tc(q)
    out_ref = jax.new_ref(out0)
    _sc_scatter(si_e, sv_e, out_ref)
    return jax.freeze(out_ref)


# speedup vs baseline: 2.6541x; 2.6541x over previous
import jax
import jax.numpy as jnp
from jax import lax
from jax.experimental import pallas as pl
from jax.experimental.pallas import tpu as pltpu
from jax.experimental.pallas import tpu_sc as plsc

_N = 16777216
_NB = 2097152
_NC = 2
_NS = 16
_NW = _NC * _NS
_C = 32768
_NCHUNK = _N // _C
_CPW = _NCHUNK // _NW
_SEG = 2048
_PAD = 2080
_NBND = 520


def _muladd_body(lam_ref, m_ref, c_ref, o_ref):
    o_ref[...] = m_ref[...] * lam_ref[0] + c_ref[...]


def _values_tc(lam, m, c):
    nblk = 8
    return pl.pallas_call(
        _muladd_body,
        grid=(nblk,),
        in_specs=[
            pl.BlockSpec(memory_space=pltpu.SMEM),
            pl.BlockSpec((_NB // nblk,), lambda i: (i,)),
            pl.BlockSpec((_NB // nblk,), lambda i: (i,)),
        ],
        out_specs=pl.BlockSpec((_NB // nblk,), lambda i: (i,)),
        out_shape=jax.ShapeDtypeStruct((_NB,), jnp.float32),
    )(lam, m, c)


def _sc_body(si_e, sv_e, bnd, q, out, img, kseg, vseg, bvm, sem):
    cid = lax.axis_index("c")
    sid = lax.axis_index("s")
    wid = sid * _NC + cid

    pltpu.sync_copy(bnd.at[pl.ds(0, _NBND)], bvm)
    iota = lax.iota(jnp.int32, 16)

    def scal(i):
        g = pl.multiple_of((i // 16) * 16, 16)
        vec = bvm[pl.ds(g, 16)]
        sel = jnp.where(iota == (i - g), vec, jnp.int32(-2147483647))
        return jnp.max(sel)

    def chunk(j, carry):
        c = wid * _CPW + j
        cbase = pl.multiple_of(c * _C, _C)
        lo = scal(c)
        hi = scal(c + 1)
        lo8 = lo & ~7
        npiece = (hi - lo8 + _SEG - 1) // _SEG

        in_img = pltpu.async_copy(q.at[pl.ds(cbase, _C)], img, sem)

        def piece(p, carry2):
            pbase = pl.multiple_of(lo8 + p * _SEG, 8)
            pltpu.sync_copy(si_e.at[pl.ds(pbase, _SEG)], kseg)
            pltpu.sync_copy(sv_e.at[pl.ds(pbase, _SEG)], vseg)
            for g in range(_SEG // 16):
                kvec = kseg[pl.ds(g * 16, 16)]
                vvec = vseg[pl.ds(g * 16, 16)]
                pos = (pbase + g * 16) + iota
                valid = (pos >= lo) & (pos < hi)
                local = jnp.where(valid, kvec - cbase, 0)
                plsc.store_scatter(img, [local], vvec, mask=valid)
            return carry2

        in_img.wait()
        lax.fori_loop(0, npiece, piece, 0)
        pltpu.sync_copy(img, out.at[pl.ds(cbase, _C)])
        return carry

    lax.fori_loop(0, _CPW, chunk, 0)


def _sc_apply(si_e, sv_e, bnd, q):
    mesh = plsc.VectorSubcoreMesh(
        core_axis_name="c", subcore_axis_name="s",
        num_cores=_NC, num_subcores=_NS)
    return pl.kernel(
        _sc_body,
        out_type=jax.ShapeDtypeStruct((_N,), jnp.float32),
        mesh=mesh,
        compiler_params=pltpu.CompilerParams(needs_layout_passes=False),
        scratch_types=[
            pltpu.VMEM((_C,), jnp.float32),
            pltpu.VMEM((_SEG,), jnp.int32),
            pltpu.VMEM((_SEG,), jnp.float32),
            pltpu.VMEM((_NBND,), jnp.int32),
            pltpu.SemaphoreType.DMA,
        ],
    )(si_e, sv_e, bnd, q)


def kernel(q, _lambda, idx_b, xb_m, xb_c):
    idx = idx_b.astype(jnp.int32)
    vals = _values_tc(_lambda, xb_m, xb_c)
    si, sv = lax.sort_key_val(idx, vals, is_stable=False)
    si_e = jnp.concatenate([si, jnp.full((_PAD,), -1, jnp.int32)])
    sv_e = jnp.concatenate([sv, jnp.zeros((_PAD,), jnp.float32)])
    edges = jnp.arange(_NCHUNK + 1, dtype=jnp.int32) * _C
    bnd = jnp.searchsorted(si, edges, side="left").astype(jnp.int32)
    bnd = jnp.concatenate(
        [bnd, jnp.zeros((_NBND - _NCHUNK - 1,), jnp.int32)])
    return _sc_apply(si_e, sv_e, bnd, q)


# speedup vs baseline: 3.0795x; 1.1603x over previous
import jax
import jax.numpy as jnp
from jax import lax
from jax.experimental import pallas as pl
from jax.experimental.pallas import tpu as pltpu
from jax.experimental.pallas import tpu_sc as plsc

_N = 16777216
_NB = 2097152
_NC = 2
_NS = 16
_NW = _NC * _NS
_C = 32768
_NCHUNK = _N // _C
_CPW = _NCHUNK // _NW
_SEG = 2048
_PAD = 2080
_NBND = 520


def _muladd_body(lam_ref, m_ref, c_ref, o_ref):
    o_ref[...] = m_ref[...] * lam_ref[0] + c_ref[...]


def _values_tc(lam, m, c):
    nblk = 8
    return pl.pallas_call(
        _muladd_body,
        grid=(nblk,),
        in_specs=[
            pl.BlockSpec(memory_space=pltpu.SMEM),
            pl.BlockSpec((_NB // nblk,), lambda i: (i,)),
            pl.BlockSpec((_NB // nblk,), lambda i: (i,)),
        ],
        out_specs=pl.BlockSpec((_NB // nblk,), lambda i: (i,)),
        out_shape=jax.ShapeDtypeStruct((_NB,), jnp.float32),
    )(lam, m, c)


def _hist_body(idx, out, kst, histf, sem):
    cid = lax.axis_index("c")
    sid = lax.axis_index("s")
    wid = sid * _NC + cid
    base = pl.multiple_of(wid * (_NB // _NW), _NB // _NW)
    iota = lax.iota(jnp.int32, 16)
    ones = jnp.ones((16,), jnp.int32)
    zeros = jnp.zeros((16,), jnp.int32)
    for z in range(512):
        histf[pl.ds(z * 16, 16)] = zeros

    def outer(it, carry):
        off = pl.multiple_of(base + it * 8192, 8192)
        pltpu.sync_copy(idx.at[pl.ds(off, 8192)], kst)
        for g in range(512):
            kvec = kst[pl.ds(g * 16, 16)]
            slot = lax.shift_right_logical(kvec, 15) * 16 + iota
            plsc.addupdate_scatter(histf, [slot], ones)
        return carry

    lax.fori_loop(0, (_NB // _NW) // 8192, outer, 0)
    pltpu.sync_copy(histf, out.at[wid])


def _sc_hist(idx):
    mesh = plsc.VectorSubcoreMesh(
        core_axis_name="c", subcore_axis_name="s",
        num_cores=_NC, num_subcores=_NS)
    return pl.kernel(
        _hist_body,
        out_type=jax.ShapeDtypeStruct((_NW, 8192), jnp.int32),
        mesh=mesh,
        compiler_params=pltpu.CompilerParams(needs_layout_passes=False),
        scratch_types=[
            pltpu.VMEM((8192,), jnp.int32),
            pltpu.VMEM((8192,), jnp.int32),
            pltpu.SemaphoreType.DMA,
        ],
    )(idx)


def _sc_body(si_e, sv_e, bnd, q, out, img, kseg, vseg, bvm, sem):
    cid = lax.axis_index("c")
    sid = lax.axis_index("s")
    wid = sid * _NC + cid

    pltpu.sync_copy(bnd.at[pl.ds(0, _NBND)], bvm)
    iota = lax.iota(jnp.int32, 16)

    def scal(i):
        g = pl.multiple_of((i // 16) * 16, 16)
        vec = bvm[pl.ds(g, 16)]
        sel = jnp.where(iota == (i - g), vec, jnp.int32(-2147483647))
        return jnp.max(sel)

    def chunk(j, carry):
        c = wid * _CPW + j
        cbase = pl.multiple_of(c * _C, _C)
        lo = scal(c)
        hi = scal(c + 1)
        lo8 = lo & ~7
        npiece = (hi - lo8 + _SEG - 1) // _SEG

        in_img = pltpu.async_copy(q.at[pl.ds(cbase, _C)], img, sem)

        def piece(p, carry2):
            pbase = pl.multiple_of(lo8 + p * _SEG, 8)
            pltpu.sync_copy(si_e.at[pl.ds(pbase, _SEG)], kseg)
            pltpu.sync_copy(sv_e.at[pl.ds(pbase, _SEG)], vseg)
            for g in range(_SEG // 16):
                kvec = kseg[pl.ds(g * 16, 16)]
                vvec = vseg[pl.ds(g * 16, 16)]
                pos = (pbase + g * 16) + iota
                valid = (pos >= lo) & (pos < hi)
                local = jnp.where(valid, kvec - cbase, 0)
                plsc.store_scatter(img, [local], vvec, mask=valid)
            return carry2

        in_img.wait()
        lax.fori_loop(0, npiece, piece, 0)
        pltpu.sync_copy(img, out.at[pl.ds(cbase, _C)])
        return carry

    lax.fori_loop(0, _CPW, chunk, 0)


def _sc_apply(si_e, sv_e, bnd, q):
    mesh = plsc.VectorSubcoreMesh(
        core_axis_name="c", subcore_axis_name="s",
        num_cores=_NC, num_subcores=_NS)
    return pl.kernel(
        _sc_body,
        out_type=jax.ShapeDtypeStruct((_N,), jnp.float32),
        mesh=mesh,
        compiler_params=pltpu.CompilerParams(needs_layout_passes=False),
        scratch_types=[
            pltpu.VMEM((_C,), jnp.float32),
            pltpu.VMEM((_SEG,), jnp.int32),
            pltpu.VMEM((_SEG,), jnp.float32),
            pltpu.VMEM((_NBND,), jnp.int32),
            pltpu.SemaphoreType.DMA,
        ],
    )(si_e, sv_e, bnd, q)


def kernel(q, _lambda, idx_b, xb_m, xb_c):
    idx = idx_b.astype(jnp.int32)
    vals = _values_tc(_lambda, xb_m, xb_c)
    si, sv = lax.sort_key_val(idx, vals, is_stable=False)
    si_e = jnp.concatenate([si, jnp.full((_PAD,), -1, jnp.int32)])
    sv_e = jnp.concatenate([sv, jnp.zeros((_PAD,), jnp.float32)])
    hist = _sc_hist(idx)
    total = hist.reshape(_NW, _NCHUNK, 16).sum(axis=(0, 2), dtype=jnp.int32)
    bnd = jnp.concatenate(
        [jnp.zeros((1,), jnp.int32), jnp.cumsum(total, dtype=jnp.int32),
         jnp.zeros((_NBND - _NCHUNK - 1,), jnp.int32)])
    return _sc_apply(si_e, sv_e, bnd, q)
